# baseline (speedup 1.0000x reference)
import functools

import jax
import jax.numpy as jnp
from jax import lax
from jax.experimental import pallas as pl
from jax.experimental.pallas import tpu as pltpu
from jax.experimental.pallas import tpu_sc as plsc

EMBED_DIM = 32
BATCH = 16384
FIELDS = 26
NUM_CORES = 2
NUM_SUBCORES = 16
NUM_WORKERS = NUM_CORES * NUM_SUBCORES
BATCH_PER_WORKER = BATCH // NUM_WORKERS
CHUNKB = 16
CHUNK = CHUNKB * FIELDS
NCHUNKS = BATCH_PER_WORKER // CHUNKB
NBUF = 8
D1 = 4
D2 = NBUF - D1
NROUNDS = NCHUNKS // NBUF


def _build():
    mesh = plsc.VectorSubcoreMesh(core_axis_name="c", subcore_axis_name="s")

    scratch = (
        [pltpu.VMEM((CHUNKB, FIELDS), jnp.int32) for _ in range(NBUF)]
        + [pltpu.VMEM((CHUNKB, FIELDS, EMBED_DIM), jnp.float32) for _ in range(NBUF)]
        + [pltpu.SemaphoreType.DMA for _ in range(2 * NBUF)]
    )

    @functools.partial(
        pl.kernel,
        mesh=mesh,
        out_type=jax.ShapeDtypeStruct((BATCH, FIELDS, EMBED_DIM), jnp.float32),
        scratch_types=scratch,
        compiler_params=pltpu.CompilerParams(use_tc_tiling_on_sc=False),
    )
    def body(x_ref, w_ref, out_ref, *s):
        idx = s[0:NBUF]
        rows = s[NBUF:2 * NBUF]
        gsem = s[2 * NBUF:3 * NBUF]
        wsem = s[3 * NBUF:4 * NBUF]

        wid = lax.axis_index("s") * NUM_CORES + lax.axis_index("c")
        base0 = wid * BATCH_PER_WORKER

        def fire_gather(c, b):
            pltpu.sync_copy(x_ref.at[pl.ds(base0 + c * CHUNKB, CHUNKB), :], idx[b])
            for i in range(CHUNKB):
                pltpu.async_copy(w_ref.at[idx[b].at[i]], rows[b].at[i], gsem[b])

        for j in range(D1):
            fire_gather(j, j)

        def round_body(r, carry):
            for b in range(NBUF):
                c = r * NBUF + b
                for i in range(CHUNKB):
                    pltpu.make_async_copy(
                        w_ref.at[idx[b].at[i]], rows[b].at[i], gsem[b]
                    ).wait()
                pltpu.async_copy(
                    rows[b],
                    out_ref.at[pl.ds(base0 + c * CHUNKB, CHUNKB), :, :],
                    wsem[b],
                )
                b2 = (b + D1) % NBUF
                c2 = c + D1

                @pl.when(c2 >= NBUF)
                def _():
                    pltpu.make_async_copy(
                        rows[b2],
                        out_ref.at[pl.ds(base0 + (c2 - NBUF) * CHUNKB, CHUNKB), :, :],
                        wsem[b2],
                    ).wait()

                @pl.when(c2 < NCHUNKS)
                def _():
                    fire_gather(c2, b2)
            return carry

        lax.fori_loop(0, NROUNDS, round_body, 0)

        for j in range(D2):
            c = NCHUNKS - D2 + j
            b = c % NBUF
            pltpu.make_async_copy(
                rows[b],
                out_ref.at[pl.ds(base0 + c * CHUNKB, CHUNKB), :, :],
                wsem[b],
            ).wait()

    return body


_gather_kernel = _build()


def kernel(x, weight):
    return _gather_kernel(x, weight)


# speedup vs baseline: 1.0073x; 1.0073x over previous
import functools

import jax
import jax.numpy as jnp
from jax import lax
from jax.experimental import pallas as pl
from jax.experimental.pallas import tpu as pltpu
from jax.experimental.pallas import tpu_sc as plsc

EMBED_DIM = 32
BATCH = 16384
FIELDS = 26
NUM_CORES = 2
NUM_SUBCORES = 16
NUM_WORKERS = NUM_CORES * NUM_SUBCORES
BATCH_PER_WORKER = BATCH // NUM_WORKERS
NCHUNKS = FIELDS
NBUF = 6
D1 = 3
D2 = NBUF - D1


def _build():
    mesh = plsc.VectorSubcoreMesh(core_axis_name="c", subcore_axis_name="s")

    scratch = (
        [pltpu.VMEM((BATCH_PER_WORKER,), jnp.int32) for _ in range(NBUF)]
        + [pltpu.VMEM((BATCH_PER_WORKER, EMBED_DIM), jnp.float32) for _ in range(NBUF)]
        + [pltpu.SemaphoreType.DMA for _ in range(2 * NBUF)]
    )

    @functools.partial(
        pl.kernel,
        mesh=mesh,
        out_type=jax.ShapeDtypeStruct((BATCH, FIELDS, EMBED_DIM), jnp.float32),
        scratch_types=scratch,
        compiler_params=pltpu.CompilerParams(use_tc_tiling_on_sc=False),
    )
    def body(xt_ref, w_ref, out_ref, *s):
        idx = s[0:NBUF]
        rows = s[NBUF:2 * NBUF]
        gsem = s[2 * NBUF:3 * NBUF]
        wsem = s[3 * NBUF:4 * NBUF]

        wid = lax.axis_index("s") * NUM_CORES + lax.axis_index("c")
        b0 = wid * BATCH_PER_WORKER

        def fire_gather(f, b):
            pltpu.sync_copy(xt_ref.at[f, pl.ds(b0, BATCH_PER_WORKER)], idx[b])
            pltpu.async_copy(w_ref.at[idx[b]], rows[b], gsem[b])

        def wait_gather(f, b):
            pltpu.make_async_copy(w_ref.at[idx[b]], rows[b], gsem[b]).wait()

        def fire_write(f, b):
            pltpu.async_copy(
                rows[b], out_ref.at[pl.ds(b0, BATCH_PER_WORKER), f, :], wsem[b]
            )

        def wait_write(f, b):
            pltpu.make_async_copy(
                rows[b], out_ref.at[pl.ds(b0, BATCH_PER_WORKER), f, :], wsem[b]
            ).wait()

        for f in range(D1):
            fire_gather(f, f % NBUF)
        for f in range(NCHUNKS):
            b = f % NBUF
            wait_gather(f, b)
            fire_write(f, b)
            f2 = f + D1
            if f2 < NCHUNKS:
                b2 = f2 % NBUF
                if f2 - NBUF >= 0:
                    wait_write(f2 - NBUF, b2)
                fire_gather(f2, b2)
        for f in range(NCHUNKS - NBUF, NCHUNKS):
            wait_write(f, f % NBUF)

    return body


_gather_kernel = _build()


def kernel(x, weight):
    return _gather_kernel(x.T, weight)
